# Initial kernel scaffold; baseline (speedup 1.0000x reference)
import jax
import jax.numpy as jnp
from jax.experimental import pallas as pl


def kernel(bincounts, global_binixs, binixs, labels, local_cell_ix, baseline_table, differential_table):
    raise NotImplementedError("write your pallas kernel here")


# speedup vs baseline: 1.4465x; 1.4465x over previous
import functools
import math

import jax
import jax.numpy as jnp
from jax import lax
from jax.experimental import pallas as pl
from jax.experimental.pallas import tpu as pltpu
from jax.experimental.pallas import tpu_sc as plsc

N_FRAG = 16384
FPS = 64
N_CELLS = 4096
N_CLUSTERS = 16
NC, NS, L = 2, 16, 16
NW = NC * NS
B_PER_W = N_FRAG // NW
CH = 128
K_CH = B_PER_W // CH
N_GROUPS = B_PER_W // L
LOG_FPS = math.log(FPS)


def _log_vec(x):
    bits = plsc.bitcast(x, jnp.int32)
    e = ((bits >> 23) & 0xFF) - 127
    m = plsc.bitcast((bits & 0x7FFFFF) | 0x3F800000, jnp.float32)
    big = m >= 1.4142135623730951
    m = jnp.where(big, m * 0.5, m)
    e = e + big.astype(jnp.int32)
    z = (m - 1.0) / (m + 1.0)
    z2 = z * z
    p = 1.0 + z2 * (1.0 / 3.0 + z2 * (0.2 + z2 * (1.0 / 7.0)))
    return e.astype(jnp.float32) * 0.6931471805599453 + 2.0 * z * p


def _body(binc_hbm, gbix_hbm, bix_hbm, labels_hbm, cix_hbm, table_hbm, diff_hbm,
          out_hbm,
          idx_v, rows_v, binc_v, labels_v, diff_v, cix_v, bix_v, y_buf, out_v,
          sem):
    wid = lax.axis_index("s") * NC + lax.axis_index("c")
    base = wid * B_PER_W

    pltpu.sync_copy(gbix_hbm.at[wid], idx_v)
    gathers = [
        pltpu.async_copy(table_hbm.at[idx_v.at[k]], rows_v.at[k], sem)
        for k in range(K_CH)
    ]
    pltpu.sync_copy(binc_hbm.at[pl.ds(base, B_PER_W)], binc_v)
    pltpu.sync_copy(labels_hbm, labels_v)
    pltpu.sync_copy(diff_hbm, diff_v)
    pltpu.sync_copy(cix_hbm.at[pl.ds(base, B_PER_W)], cix_v)
    pltpu.sync_copy(bix_hbm.at[pl.ds(base, B_PER_W)], bix_v)
    for g in gathers:
        g.wait()

    iota = lax.iota(jnp.int32, L)

    def group_body(g, carry):
        f = g * L + iota
        k_vec = lax.shift_right_logical(f, 7)
        r_vec = f & (CH - 1)
        cix = cix_v[pl.ds(g * L, L)]
        clu = plsc.load_gather(labels_v, [cix])
        w = plsc.load_gather(diff_v, [clu])

        def j_body(j, m):
            jv = jnp.full((L,), j, jnp.int32)
            bse = plsc.load_gather(rows_v, [k_vec, r_vec, jv])
            bc = plsc.load_gather(binc_v, [f, jv])
            y = bse + jnp.where(bc > 1, w, 0.0)
            y_buf[j] = y
            return jnp.maximum(m, y)

        m = lax.fori_loop(0, FPS, j_body, jnp.full((L,), -1e30, jnp.float32))

        def j_body2(j, s):
            return s + jnp.exp(y_buf[j] - m)

        s = lax.fori_loop(0, FPS, j_body2, jnp.zeros((L,), jnp.float32))

        bix = bix_v[pl.ds(g * L, L)]
        yp = plsc.load_gather(y_buf, [bix, iota])
        out_v[pl.ds(g * L, L)] = yp - m - _log_vec(s) + LOG_FPS
        return carry

    lax.fori_loop(0, N_GROUPS, group_body, 0)
    pltpu.sync_copy(out_v, out_hbm.at[pl.ds(base, B_PER_W)])


@functools.cache
def _make_sc_call():
    mesh = plsc.VectorSubcoreMesh(
        core_axis_name="c", subcore_axis_name="s",
        num_cores=NC, num_subcores=NS)
    return pl.kernel(
        _body,
        out_type=jax.ShapeDtypeStruct((N_FRAG,), jnp.float32),
        mesh=mesh,
        scratch_types=[
            pltpu.VMEM((K_CH, CH), jnp.int32),
            pltpu.VMEM((K_CH, CH, FPS), jnp.float32),
            pltpu.VMEM((B_PER_W, FPS), jnp.int32),
            pltpu.VMEM((N_CELLS,), jnp.int32),
            pltpu.VMEM((N_CLUSTERS,), jnp.float32),
            pltpu.VMEM((B_PER_W,), jnp.int32),
            pltpu.VMEM((B_PER_W,), jnp.int32),
            pltpu.VMEM((FPS, L), jnp.float32),
            pltpu.VMEM((B_PER_W,), jnp.float32),
            pltpu.SemaphoreType.DMA,
        ],
        compiler_params=pltpu.CompilerParams(
            needs_layout_passes=False, use_tc_tiling_on_sc=False),
    )


def kernel(bincounts, global_binixs, binixs, labels, local_cell_ix,
           baseline_table, differential_table):
    gbix = global_binixs.reshape(NW, K_CH, CH)
    bix = binixs.reshape(N_FRAG)
    diff = differential_table.reshape(N_CLUSTERS)
    return _make_sc_call()(bincounts, gbix, bix, labels, local_cell_ix,
                           baseline_table, diff)


# speedup vs baseline: 1.4925x; 1.0318x over previous
import functools
import math

import jax
import jax.numpy as jnp
from jax import lax
from jax.experimental import pallas as pl
from jax.experimental.pallas import tpu as pltpu
from jax.experimental.pallas import tpu_sc as plsc

N_FRAG = 16384
FPS = 64
N_CELLS = 4096
N_CLUSTERS = 16
NC, NS, L = 2, 16, 16
NW = NC * NS
B_PER_W = N_FRAG // NW
CH = 128
K_CH = B_PER_W // CH
N_GROUPS = B_PER_W // L
LOG_FPS = math.log(FPS)


def _log_vec(x):
    bits = plsc.bitcast(x, jnp.int32)
    e = ((bits >> 23) & 0xFF) - 127
    m = plsc.bitcast((bits & 0x7FFFFF) | 0x3F800000, jnp.float32)
    big = m >= 1.4142135623730951
    m = jnp.where(big, m * 0.5, m)
    e = e + big.astype(jnp.int32)
    z = (m - 1.0) / (m + 1.0)
    z2 = z * z
    p = 1.0 + z2 * (1.0 / 3.0 + z2 * (0.2 + z2 * (1.0 / 7.0)))
    return e.astype(jnp.float32) * 0.6931471805599453 + 2.0 * z * p


def _body(binc_hbm, gbix_hbm, bix_hbm, labels_hbm, cix_hbm, table_hbm, diff_hbm,
          out_hbm,
          idx_v, rows_v, binc_v, labels_v, diff_v, cix_v, bix_v, y_buf, out_v,
          sem):
    wid = lax.axis_index("s") * NC + lax.axis_index("c")
    base = wid * B_PER_W

    pltpu.sync_copy(gbix_hbm.at[wid], idx_v)
    gathers = [
        pltpu.async_copy(table_hbm.at[idx_v.at[k]], rows_v.at[k], sem)
        for k in range(K_CH)
    ]
    pltpu.sync_copy(binc_hbm.at[pl.ds(base, B_PER_W)], binc_v)
    pltpu.sync_copy(labels_hbm, labels_v)
    pltpu.sync_copy(diff_hbm, diff_v)
    pltpu.sync_copy(cix_hbm.at[pl.ds(base, B_PER_W)], cix_v)
    pltpu.sync_copy(bix_hbm.at[pl.ds(base, B_PER_W)], bix_v)
    for g in gathers:
        g.wait()

    iota = lax.iota(jnp.int32, L)

    def group_body(g, carry):
        f = g * L + iota
        k_vec = lax.shift_right_logical(f, 7)
        r_vec = f & (CH - 1)
        cix = cix_v[pl.ds(g * L, L)]
        clu = plsc.load_gather(labels_v, [cix])
        w = plsc.load_gather(diff_v, [clu])

        m = None
        for j in range(FPS):
            jv = jnp.full((L,), j, jnp.int32)
            bse = plsc.load_gather(rows_v, [k_vec, r_vec, jv])
            bc = plsc.load_gather(binc_v, [f, jv])
            y = bse + jnp.where(bc > 1, w, 0.0)
            y_buf[j] = y
            m = y if m is None else jnp.maximum(m, y)

        s = jnp.zeros((L,), jnp.float32)
        for j in range(FPS):
            s = s + jnp.exp(y_buf[j] - m)

        bix = bix_v[pl.ds(g * L, L)]
        yp = plsc.load_gather(y_buf, [bix, iota])
        out_v[pl.ds(g * L, L)] = yp - m - _log_vec(s) + LOG_FPS
        return carry

    lax.fori_loop(0, N_GROUPS, group_body, 0)
    pltpu.sync_copy(out_v, out_hbm.at[pl.ds(base, B_PER_W)])


@functools.cache
def _make_sc_call():
    mesh = plsc.VectorSubcoreMesh(
        core_axis_name="c", subcore_axis_name="s",
        num_cores=NC, num_subcores=NS)
    return pl.kernel(
        _body,
        out_type=jax.ShapeDtypeStruct((N_FRAG,), jnp.float32),
        mesh=mesh,
        scratch_types=[
            pltpu.VMEM((K_CH, CH), jnp.int32),
            pltpu.VMEM((K_CH, CH, FPS), jnp.float32),
            pltpu.VMEM((B_PER_W, FPS), jnp.int32),
            pltpu.VMEM((N_CELLS,), jnp.int32),
            pltpu.VMEM((N_CLUSTERS,), jnp.float32),
            pltpu.VMEM((B_PER_W,), jnp.int32),
            pltpu.VMEM((B_PER_W,), jnp.int32),
            pltpu.VMEM((FPS, L), jnp.float32),
            pltpu.VMEM((B_PER_W,), jnp.float32),
            pltpu.SemaphoreType.DMA,
        ],
        compiler_params=pltpu.CompilerParams(
            needs_layout_passes=False, use_tc_tiling_on_sc=False),
    )


def kernel(bincounts, global_binixs, binixs, labels, local_cell_ix,
           baseline_table, differential_table):
    gbix = global_binixs.reshape(NW, K_CH, CH)
    bix = binixs.reshape(N_FRAG)
    diff = differential_table.reshape(N_CLUSTERS)
    return _make_sc_call()(bincounts, gbix, bix, labels, local_cell_ix,
                           baseline_table, diff)
